# baseline (speedup 1.0000x reference)
import functools

import jax
import jax.numpy as jnp
from jax import lax
from jax.experimental import pallas as pl
from jax.experimental.pallas import tpu as pltpu
from jax.experimental.pallas import tpu_sc as plsc

EF = 128
KNN = 8

_NC = 2
_NS = 16
_NW = _NC * _NS

_CH = 448
_CPW = 28


def _h_matmul_body(x_ref, w_ref, o_ref):
    o_ref[...] = jnp.dot(x_ref[...], w_ref[...],
                         preferred_element_type=jnp.float32)


def _h_matmul(x, w_t):
    n = x.shape[0]
    bm = 5000
    grid = n // bm
    return pl.pallas_call(
        _h_matmul_body,
        grid=(grid,),
        in_specs=[
            pl.BlockSpec((bm, EF), lambda i: (i, 0)),
            pl.BlockSpec((EF, EF), lambda i: (0, 0)),
        ],
        out_specs=pl.BlockSpec((bm, EF), lambda i: (i, 0)),
        out_shape=jax.ShapeDtypeStruct((n, EF), jnp.float32),
    )(x, w_t)


def _sc_gather_body(h_hbm, idx_hbm, out_hbm, idx_v0, idx_v1, rows_v,
                    gsem, ssem):
    wid = lax.axis_index("s") * _NC + lax.axis_index("c")
    base = wid * (_CH * _CPW)
    idx_v = [idx_v0, idx_v1]

    store = [None, None]
    gath = [None, None]

    def start(i):
        b = i % 2
        pltpu.sync_copy(idx_hbm.at[pl.ds(base + i * _CH, _CH)], idx_v[b])
        gath[b] = pltpu.async_copy(h_hbm.at[idx_v[b]], rows_v.at[b],
                                   gsem.at[b])

    start(0)
    for i in range(_CPW):
        b = i % 2
        if i + 1 < _CPW:
            if store[1 - b] is not None:
                store[1 - b].wait()
            start(i + 1)
        gath[b].wait()
        store[b] = pltpu.async_copy(
            rows_v.at[b], out_hbm.at[pl.ds(base + i * _CH, _CH)], ssem.at[b])
    store[0].wait()
    store[1].wait()


def _sc_gather(h, idx_pad, e_pad):
    mesh = plsc.VectorSubcoreMesh(core_axis_name="c", subcore_axis_name="s")
    k = pl.kernel(
        _sc_gather_body,
        out_type=jax.ShapeDtypeStruct((e_pad, EF), jnp.float32),
        mesh=mesh,
        scratch_types=[
            pltpu.VMEM((_CH,), jnp.int32),
            pltpu.VMEM((_CH,), jnp.int32),
            pltpu.VMEM((2, _CH, EF), jnp.float32),
            pltpu.SemaphoreType.DMA((2,)),
            pltpu.SemaphoreType.DMA((2,)),
        ],
    )
    return k(h, idx_pad)


def _mlp_max_body(g_ref, xyz_ref, w1x_ref, w2t_ref, b2_ref, o_ref):
    bn = o_ref.shape[0]
    xyzc = jnp.dot(xyz_ref[...].reshape(KNN * bn, 8), w1x_ref[...],
                   preferred_element_type=jnp.float32).reshape(KNN, bn, EF)
    acc = None
    for k in range(KNN):
        pre = g_ref[k] + xyzc[k]
        act = jnp.where(pre >= 0, pre, 0.01 * pre)
        o2 = jnp.dot(act.astype(jnp.bfloat16), w2t_ref[...],
                     preferred_element_type=jnp.float32)
        acc = o2 if acc is None else jnp.maximum(acc, o2)
    o_ref[...] = acc + b2_ref[...]


def _mlp_max(g3, xyz3, w1x8, w2_t, b2, n_nodes):
    bn = 1000
    grid = n_nodes // bn
    return pl.pallas_call(
        _mlp_max_body,
        grid=(grid,),
        in_specs=[
            pl.BlockSpec((KNN, bn, EF), lambda i: (0, i, 0)),
            pl.BlockSpec((KNN, bn, 8), lambda i: (0, i, 0)),
            pl.BlockSpec((8, EF), lambda i: (0, 0)),
            pl.BlockSpec((EF, EF), lambda i: (0, 0)),
            pl.BlockSpec((1, EF), lambda i: (0, 0)),
        ],
        out_specs=pl.BlockSpec((bn, EF), lambda i: (i, 0)),
        out_shape=jax.ShapeDtypeStruct((n_nodes, EF), jnp.float32),
    )(g3, xyz3, w1x8, w2_t, b2)


def kernel(input, KNN_idx, KNN_xyz, W1, b1, W2, b2):
    n = input.shape[0]
    e = KNN_idx.shape[0]

    idx = KNN_idx.astype(jnp.int32)
    e_pad = _NW * _CPW * _CH
    n_pad = e_pad // KNN
    idx_pad = jnp.pad(idx.reshape(n, KNN),
                      ((0, n_pad - n), (0, 0))).T.reshape(e_pad)

    w1f_t = W1[:, :EF].T
    w1x8 = (jnp.zeros((8, EF), jnp.float32)
            .at[:3].set(W1[:, EF:].T).at[3].set(b1))
    w2_t = W2.T.astype(jnp.bfloat16)

    h = _h_matmul(input, w1f_t)
    g = _sc_gather(h, idx_pad, e_pad)
    g3 = g.reshape(KNN, n_pad, EF)

    xyz3 = jnp.concatenate(
        [KNN_xyz, jnp.ones((e, 1), jnp.float32),
         jnp.zeros((e, 4), jnp.float32)],
        axis=1).reshape(n, KNN, 8).transpose(1, 0, 2)

    return _mlp_max(g3, xyz3, w1x8, w2_t, b2.reshape(1, EF), n)


# speedup vs baseline: 1.3253x; 1.3253x over previous
import functools

import jax
import jax.numpy as jnp
from jax import lax
from jax.experimental import pallas as pl
from jax.experimental.pallas import tpu as pltpu
from jax.experimental.pallas import tpu_sc as plsc

EF = 128
KNN = 8

_NC = 2
_NS = 16
_NW = _NC * _NS

_CH = 448
_CPW = 28


def _h_matmul_body(x_ref, w_ref, o_ref):
    o_ref[...] = jnp.dot(x_ref[...], w_ref[...],
                         preferred_element_type=jnp.float32)


def _h_matmul(x, w_t):
    n = x.shape[0]
    bm = 5000
    grid = n // bm
    return pl.pallas_call(
        _h_matmul_body,
        grid=(grid,),
        in_specs=[
            pl.BlockSpec((bm, EF), lambda i: (i, 0)),
            pl.BlockSpec((EF, EF), lambda i: (0, 0)),
        ],
        out_specs=pl.BlockSpec((bm, EF), lambda i: (i, 0)),
        out_shape=jax.ShapeDtypeStruct((n, EF), jnp.float32),
    )(x, w_t)


def _sc_gather_body(h_hbm, idx_hbm, out_hbm, idx_v0, idx_v1, rows_v,
                    gsem, ssem):
    wid = lax.axis_index("s") * _NC + lax.axis_index("c")
    base = wid * (_CH * _CPW)
    idx_v = [idx_v0, idx_v1]

    store = [None, None]
    gath = [None, None]

    def start(i):
        b = i % 2
        pltpu.sync_copy(idx_hbm.at[pl.ds(base + i * _CH, _CH)], idx_v[b])
        gath[b] = pltpu.async_copy(h_hbm.at[idx_v[b]], rows_v.at[b],
                                   gsem.at[b])

    start(0)
    for i in range(_CPW):
        b = i % 2
        if i + 1 < _CPW:
            if store[1 - b] is not None:
                store[1 - b].wait()
            start(i + 1)
        gath[b].wait()
        store[b] = pltpu.async_copy(
            rows_v.at[b], out_hbm.at[pl.ds(base + i * _CH, _CH)], ssem.at[b])
    store[0].wait()
    store[1].wait()


def _sc_gather(h, idx_pad, e_pad):
    mesh = plsc.VectorSubcoreMesh(core_axis_name="c", subcore_axis_name="s")
    k = pl.kernel(
        _sc_gather_body,
        out_type=jax.ShapeDtypeStruct((e_pad, EF), jnp.float32),
        mesh=mesh,
        scratch_types=[
            pltpu.VMEM((_CH,), jnp.int32),
            pltpu.VMEM((_CH,), jnp.int32),
            pltpu.VMEM((2, _CH, EF), jnp.float32),
            pltpu.SemaphoreType.DMA((2,)),
            pltpu.SemaphoreType.DMA((2,)),
        ],
    )
    return k(h, idx_pad)


def _mlp_max_body(g_ref, xyz_ref, w1x_ref, w2t_ref, b2_ref, o_ref):
    pre = g_ref[...] + jnp.dot(xyz_ref[...], w1x_ref[...],
                               preferred_element_type=jnp.float32)
    act = jnp.where(pre >= 0, pre, 0.01 * pre)
    o2 = jnp.dot(act.astype(jnp.bfloat16), w2t_ref[...],
                 preferred_element_type=jnp.float32)
    bm = o2.shape[0]
    o_ref[...] = jnp.max(o2.reshape(bm // KNN, KNN, EF), axis=1) + b2_ref[...]


def _mlp_max(g, xyz8, w1x8, w2_t, b2, n_nodes):
    e = n_nodes * KNN
    bm = 8000
    grid = e // bm
    return pl.pallas_call(
        _mlp_max_body,
        grid=(grid,),
        in_specs=[
            pl.BlockSpec((bm, EF), lambda i: (i, 0)),
            pl.BlockSpec((bm, 8), lambda i: (i, 0)),
            pl.BlockSpec((8, EF), lambda i: (0, 0)),
            pl.BlockSpec((EF, EF), lambda i: (0, 0)),
            pl.BlockSpec((1, EF), lambda i: (0, 0)),
        ],
        out_specs=pl.BlockSpec((bm // KNN, EF), lambda i: (i, 0)),
        out_shape=jax.ShapeDtypeStruct((n_nodes, EF), jnp.float32),
    )(g, xyz8, w1x8, w2_t, b2)


def kernel(input, KNN_idx, KNN_xyz, W1, b1, W2, b2):
    n = input.shape[0]
    e = KNN_idx.shape[0]

    idx = KNN_idx.astype(jnp.int32)
    e_pad = _NW * _CPW * _CH
    idx_pad = jnp.concatenate(
        [idx, jnp.zeros((e_pad - e,), dtype=jnp.int32)])

    w1f_t = W1[:, :EF].T
    w1x8 = (jnp.zeros((8, EF), jnp.float32)
            .at[:3].set(W1[:, EF:].T).at[3].set(b1))
    w2_t = W2.T.astype(jnp.bfloat16)

    h = _h_matmul(input, w1f_t)
    g = _sc_gather(h, idx_pad, e_pad)

    xyz8 = jnp.concatenate(
        [KNN_xyz, jnp.ones((e, 1), jnp.float32),
         jnp.zeros((e, 4), jnp.float32)], axis=1)

    return _mlp_max(g, xyz8, w1x8, w2_t, b2.reshape(1, EF), n)


# speedup vs baseline: 1.3453x; 1.0151x over previous
import functools

import jax
import jax.numpy as jnp
from jax import lax
from jax.experimental import pallas as pl
from jax.experimental.pallas import tpu as pltpu
from jax.experimental.pallas import tpu_sc as plsc

EF = 128
KNN = 8

_NC = 2
_NS = 16
_NW = _NC * _NS

_CH = 448
_CPW = 28


def _h_matmul_body(x_ref, w_ref, o_ref):
    o_ref[...] = jnp.dot(x_ref[...], w_ref[...],
                         preferred_element_type=jnp.float32)


def _h_matmul(x, w_t):
    n = x.shape[0]
    bm = 5000
    grid = n // bm
    return pl.pallas_call(
        _h_matmul_body,
        grid=(grid,),
        in_specs=[
            pl.BlockSpec((bm, EF), lambda i: (i, 0)),
            pl.BlockSpec((EF, EF), lambda i: (0, 0)),
        ],
        out_specs=pl.BlockSpec((bm, EF), lambda i: (i, 0)),
        out_shape=jax.ShapeDtypeStruct((n, EF), jnp.float32),
    )(x, w_t)


def _sc_gather_body(h_hbm, idx_hbm, out_hbm, idx_v0, idx_v1, rows_v,
                    gsem, ssem):
    wid = lax.axis_index("s") * _NC + lax.axis_index("c")
    base = wid * (_CH * _CPW)
    idx_v = [idx_v0, idx_v1]

    store = [None, None]
    gath = [None, None]

    def start(i):
        b = i % 2
        pltpu.sync_copy(idx_hbm.at[pl.ds(base + i * _CH, _CH)], idx_v[b])
        gath[b] = pltpu.async_copy(h_hbm.at[idx_v[b]], rows_v.at[b],
                                   gsem.at[b])

    start(0)
    for i in range(_CPW):
        b = i % 2
        if i + 1 < _CPW:
            if store[1 - b] is not None:
                store[1 - b].wait()
            start(i + 1)
        gath[b].wait()
        store[b] = pltpu.async_copy(
            rows_v.at[b], out_hbm.at[pl.ds(base + i * _CH, _CH)], ssem.at[b])
    store[0].wait()
    store[1].wait()


def _sc_gather(h, idx_pad, e_pad):
    mesh = plsc.VectorSubcoreMesh(core_axis_name="c", subcore_axis_name="s")
    k = pl.kernel(
        _sc_gather_body,
        out_type=jax.ShapeDtypeStruct((e_pad, EF), jnp.float32),
        mesh=mesh,
        scratch_types=[
            pltpu.VMEM((_CH,), jnp.int32),
            pltpu.VMEM((_CH,), jnp.int32),
            pltpu.VMEM((2, _CH, EF), jnp.float32),
            pltpu.SemaphoreType.DMA((2,)),
            pltpu.SemaphoreType.DMA((2,)),
        ],
    )
    return k(h, idx_pad)


def _mlp_max_body(g_ref, xyz_ref, w1x_ref, b1_ref, w2t_ref, b2_ref, o_ref):
    pre = (g_ref[...] + b1_ref[...]
           + jnp.dot(xyz_ref[...], w1x_ref[...],
                     preferred_element_type=jnp.float32))
    act = jnp.where(pre >= 0, pre, 0.01 * pre)
    o2 = jnp.dot(act.astype(jnp.bfloat16), w2t_ref[...],
                 preferred_element_type=jnp.float32)
    bm = o2.shape[0]
    o_ref[...] = jnp.max(o2.reshape(bm // KNN, KNN, EF), axis=1) + b2_ref[...]


def _mlp_max(g, xyz, w1x_t, b1, w2_t, b2, n_nodes):
    e = n_nodes * KNN
    bm = 8000
    grid = e // bm
    return pl.pallas_call(
        _mlp_max_body,
        grid=(grid,),
        in_specs=[
            pl.BlockSpec((bm, EF), lambda i: (i, 0)),
            pl.BlockSpec((bm, 3), lambda i: (i, 0)),
            pl.BlockSpec((3, EF), lambda i: (0, 0)),
            pl.BlockSpec((1, EF), lambda i: (0, 0)),
            pl.BlockSpec((EF, EF), lambda i: (0, 0)),
            pl.BlockSpec((1, EF), lambda i: (0, 0)),
        ],
        out_specs=pl.BlockSpec((bm // KNN, EF), lambda i: (i, 0)),
        out_shape=jax.ShapeDtypeStruct((n_nodes, EF), jnp.float32),
    )(g, xyz, w1x_t, b1, w2_t, b2)


def kernel(input, KNN_idx, KNN_xyz, W1, b1, W2, b2):
    n = input.shape[0]
    e = KNN_idx.shape[0]

    idx = KNN_idx.astype(jnp.int32)
    e_pad = _NW * _CPW * _CH
    idx_pad = jnp.concatenate(
        [idx, jnp.zeros((e_pad - e,), dtype=jnp.int32)])

    w1f_t = W1[:, :EF].T
    w1x_t = W1[:, EF:].T
    w2_t = W2.T.astype(jnp.bfloat16)

    h = _h_matmul(input, w1f_t)
    g = _sc_gather(h, idx_pad, e_pad)

    return _mlp_max(g, KNN_xyz, w1x_t, b1.reshape(1, EF), w2_t,
                    b2.reshape(1, EF), n)


# speedup vs baseline: 1.6996x; 1.2634x over previous
import functools

import jax
import jax.numpy as jnp
from jax import lax
from jax.experimental import pallas as pl
from jax.experimental.pallas import tpu as pltpu
from jax.experimental.pallas import tpu_sc as plsc

EF = 128
KNN = 8

_NC = 2
_NS = 16
_NW = _NC * _NS

_CH = 448
_CPW = 28
_CPW0 = 33
_CPW1 = 23


def _h_matmul_body(x_ref, w_ref, o_ref):
    o_ref[...] = jnp.dot(x_ref[...], w_ref[...],
                         preferred_element_type=jnp.float32)


def _h_matmul(x, w_t):
    n = x.shape[0]
    bm = 5000
    grid = n // bm
    return pl.pallas_call(
        _h_matmul_body,
        grid=(grid,),
        in_specs=[
            pl.BlockSpec((bm, EF), lambda i: (i, 0)),
            pl.BlockSpec((EF, EF), lambda i: (0, 0)),
        ],
        out_specs=pl.BlockSpec((bm, EF), lambda i: (i, 0)),
        out_shape=jax.ShapeDtypeStruct((n, EF), jnp.float32),
    )(x, w_t)


def _sc_gather_body(h_hbm, idx_hbm, out_hbm, idx_v0, idx_v1, rows_v,
                    gsem, ssem):
    c = lax.axis_index("c")
    s = lax.axis_index("s")
    cpw = jnp.where(c == 0, _CPW0, _CPW1)
    base = jnp.where(c == 0, s * (_CH * _CPW0),
                     _NS * (_CH * _CPW0) + s * (_CH * _CPW1))
    idx_v = [idx_v0, idx_v1]

    def wait_g(b):
        pltpu.make_async_copy(h_hbm.at[pl.ds(0, _CH)], rows_v.at[b],
                              gsem.at[b]).wait()

    def wait_s(b):
        pltpu.make_async_copy(rows_v.at[b], out_hbm.at[pl.ds(0, _CH)],
                              ssem.at[b]).wait()

    def start(i):
        b = i % 2

        @pl.when(i < cpw)
        def _():
            pltpu.sync_copy(idx_hbm.at[pl.ds(base + i * _CH, _CH)], idx_v[b])
            pltpu.async_copy(h_hbm.at[idx_v[b]], rows_v.at[b], gsem.at[b])

    start(0)
    for i in range(_CPW0):
        b = i % 2
        if i + 1 < _CPW0:
            if i >= 1:
                @pl.when(i + 1 < cpw)
                def _(b=b):
                    wait_s(1 - b)
            start(i + 1)

        @pl.when(i < cpw)
        def _(b=b, i=i):
            wait_g(b)
            pltpu.async_copy(rows_v.at[b],
                             out_hbm.at[pl.ds(base + i * _CH, _CH)],
                             ssem.at[b])
    wait_s(0)
    wait_s(1)


def _sc_gather(h, idx_pad, e_pad):
    mesh = plsc.VectorSubcoreMesh(core_axis_name="c", subcore_axis_name="s")
    k = pl.kernel(
        _sc_gather_body,
        out_type=jax.ShapeDtypeStruct((e_pad, EF), jnp.float32),
        mesh=mesh,
        scratch_types=[
            pltpu.VMEM((_CH,), jnp.int32),
            pltpu.VMEM((_CH,), jnp.int32),
            pltpu.VMEM((2, _CH, EF), jnp.float32),
            pltpu.SemaphoreType.DMA((2,)),
            pltpu.SemaphoreType.DMA((2,)),
        ],
    )
    return k(h, idx_pad)


def _mlp_max_body(g_ref, xyz_ref, w1x_ref, b1_ref, w2t_ref, b2_ref, o_ref):
    xyzc = lax.dot_general(xyz_ref[0], w1x_ref[...],
                           dimension_numbers=(((0,), (0,)), ((), ())),
                           preferred_element_type=jnp.float32)
    pre = g_ref[...] + b1_ref[...] + xyzc
    act = jnp.where(pre >= 0, pre, 0.01 * pre)
    o2 = jnp.dot(act.astype(jnp.bfloat16), w2t_ref[...],
                 preferred_element_type=jnp.float32)
    bm = o2.shape[0]
    o_ref[...] = jnp.max(o2.reshape(bm // KNN, KNN, EF), axis=1) + b2_ref[...]


def _mlp_max(g, xyz_t, w1x_t, b1, w2_t, b2, n_nodes):
    e = n_nodes * KNN
    bm = 8000
    grid = e // bm
    return pl.pallas_call(
        _mlp_max_body,
        grid=(grid,),
        in_specs=[
            pl.BlockSpec((bm, EF), lambda i: (i, 0)),
            pl.BlockSpec((1, 3, bm), lambda i: (i, 0, 0)),
            pl.BlockSpec((3, EF), lambda i: (0, 0)),
            pl.BlockSpec((1, EF), lambda i: (0, 0)),
            pl.BlockSpec((EF, EF), lambda i: (0, 0)),
            pl.BlockSpec((1, EF), lambda i: (0, 0)),
        ],
        out_specs=pl.BlockSpec((bm // KNN, EF), lambda i: (i, 0)),
        out_shape=jax.ShapeDtypeStruct((n_nodes, EF), jnp.float32),
    )(g, xyz_t, w1x_t, b1, w2_t, b2)


def kernel(input, KNN_idx, KNN_xyz, W1, b1, W2, b2):
    n = input.shape[0]
    e = KNN_idx.shape[0]

    idx = KNN_idx.astype(jnp.int32)
    e_pad = _NW * _CPW * _CH
    idx_pad = jnp.concatenate(
        [idx, jnp.zeros((e_pad - e,), dtype=jnp.int32)])

    w1f_t = W1[:, :EF].T
    w1x_t = W1[:, EF:].T
    w2_t = W2.T.astype(jnp.bfloat16)

    h = _h_matmul(input, w1f_t)
    g = _sc_gather(h, idx_pad, e_pad)

    bm = 8000
    xyz_t = KNN_xyz.reshape(e // bm, bm, 3).transpose(0, 2, 1)

    return _mlp_max(g, xyz_t, w1x_t, b1.reshape(1, EF), w2_t,
                    b2.reshape(1, EF), n)
